# baseline (speedup 1.0000x reference)
import functools

import jax
import jax.numpy as jnp
from jax.experimental import pallas as pl
from jax.experimental.pallas import tpu as pltpu

_ROWS = 16384
_COLS = 128
_STREAMS = 8
_BLOCK_ROWS = 512
_GRID = _ROWS // (_STREAMS * _BLOCK_ROWS)
_LOG2E = 1.4426950408889634


def _block_stats(x):
    bm = jnp.max(x, axis=0, keepdims=True)
    e = jnp.exp2(x * _LOG2E - bm * _LOG2E)
    ones = jnp.ones((1, x.shape[0]), jnp.float32)
    bs = jnp.dot(ones, e, preferred_element_type=jnp.float32)
    bt = jnp.dot(ones, x, preferred_element_type=jnp.float32)
    return bm, bs, bt


def _merge(m1, s1, m2, s2):
    m = jnp.maximum(m1, m2)
    s = (s1 * jnp.exp2((m1 - m) * _LOG2E)
         + s2 * jnp.exp2((m2 - m) * _LOG2E))
    return m, s


def _listmle_body(*refs):
    x_refs = refs[:_STREAMS]
    out_ref, m_ref, s_ref, t_ref = refs[_STREAMS:]
    i = pl.program_id(0)

    stats = [_block_stats(x_ref[...]) for x_ref in x_refs]
    m_all, s_all, t_all = stats[0]
    for bm, bs, bt in stats[1:]:
        m_all, s_all = _merge(m_all, s_all, bm, bs)
        t_all = t_all + bt

    @pl.when(i == 0)
    def _init():
        m_ref[...] = m_all
        s_ref[...] = s_all
        t_ref[...] = t_all

    @pl.when(i > 0)
    def _update():
        m_new, s_new = _merge(m_ref[...], s_ref[...], m_all, s_all)
        m_ref[...] = m_new
        s_ref[...] = s_new
        t_ref[...] = t_ref[...] + t_all

    @pl.when(i == pl.num_programs(0) - 1)
    def _finish():
        lse = m_ref[...] + jnp.log(s_ref[...])
        out_ref[...] = (_ROWS * jnp.sum(lse, keepdims=True)
                        - jnp.sum(t_ref[...], keepdims=True))


@functools.partial(jax.jit, static_argnames=())
def _listmle_loss(scores):
    out = pl.pallas_call(
        _listmle_body,
        grid=(_GRID,),
        in_specs=[
            pl.BlockSpec((_BLOCK_ROWS, _COLS), lambda i, k=k: (i + k * _GRID, 0))
            for k in range(_STREAMS)
        ],
        out_specs=pl.BlockSpec((1, 1), lambda i: (0, 0)),
        out_shape=jax.ShapeDtypeStruct((1, 1), jnp.float32),
        scratch_shapes=[
            pltpu.VMEM((1, _COLS), jnp.float32),
            pltpu.VMEM((1, _COLS), jnp.float32),
            pltpu.VMEM((1, _COLS), jnp.float32),
        ],
    )(*([scores] * _STREAMS))
    return out[0, 0]


def kernel(scores, targets):
    del targets
    return _listmle_loss(scores)


# speedup vs baseline: 1.0755x; 1.0755x over previous
import functools

import jax
import jax.numpy as jnp
from jax.experimental import pallas as pl
from jax.experimental.pallas import tpu as pltpu

_ROWS = 16384
_COLS = 128
_STREAMS = 16
_BLOCK_ROWS = 512
_GRID = _ROWS // (_STREAMS * _BLOCK_ROWS)
_LOG2E = 1.4426950408889634


def _block_stats(x):
    bm = jnp.max(x, axis=0, keepdims=True)
    e = jnp.exp2(x * _LOG2E - bm * _LOG2E)
    ones = jnp.ones((1, x.shape[0]), jnp.float32)
    bs = jnp.dot(ones, e, preferred_element_type=jnp.float32)
    bt = jnp.dot(ones, x, preferred_element_type=jnp.float32)
    return bm, bs, bt


def _merge(m1, s1, m2, s2):
    m = jnp.maximum(m1, m2)
    s = (s1 * jnp.exp2((m1 - m) * _LOG2E)
         + s2 * jnp.exp2((m2 - m) * _LOG2E))
    return m, s


def _listmle_body(*refs):
    x_refs = refs[:_STREAMS]
    out_ref, m_ref, s_ref, t_ref = refs[_STREAMS:]
    i = pl.program_id(0)

    stats = [_block_stats(x_ref[...]) for x_ref in x_refs]
    m_all, s_all, t_all = stats[0]
    for bm, bs, bt in stats[1:]:
        m_all, s_all = _merge(m_all, s_all, bm, bs)
        t_all = t_all + bt

    @pl.when(i == 0)
    def _init():
        m_ref[...] = m_all
        s_ref[...] = s_all
        t_ref[...] = t_all

    @pl.when(i > 0)
    def _update():
        m_new, s_new = _merge(m_ref[...], s_ref[...], m_all, s_all)
        m_ref[...] = m_new
        s_ref[...] = s_new
        t_ref[...] = t_ref[...] + t_all

    @pl.when(i == pl.num_programs(0) - 1)
    def _finish():
        lse = m_ref[...] + jnp.log(s_ref[...])
        out_ref[...] = (_ROWS * jnp.sum(lse, keepdims=True)
                        - jnp.sum(t_ref[...], keepdims=True))


@functools.partial(jax.jit, static_argnames=())
def _listmle_loss(scores):
    out = pl.pallas_call(
        _listmle_body,
        grid=(_GRID,),
        in_specs=[
            pl.BlockSpec((_BLOCK_ROWS, _COLS), lambda i, k=k: (i + k * _GRID, 0))
            for k in range(_STREAMS)
        ],
        out_specs=pl.BlockSpec((1, 1), lambda i: (0, 0)),
        out_shape=jax.ShapeDtypeStruct((1, 1), jnp.float32),
        scratch_shapes=[
            pltpu.VMEM((1, _COLS), jnp.float32),
            pltpu.VMEM((1, _COLS), jnp.float32),
            pltpu.VMEM((1, _COLS), jnp.float32),
        ],
    )(*([scores] * _STREAMS))
    return out[0, 0]


def kernel(scores, targets):
    del targets
    return _listmle_loss(scores)


# speedup vs baseline: 1.1158x; 1.0375x over previous
import functools

import jax
import jax.numpy as jnp
from jax.experimental import pallas as pl
from jax.experimental.pallas import tpu as pltpu

_ROWS = 16384
_COLS = 128
_STREAMS = 8
_BLOCK_ROWS = 1024
_GRID = _ROWS // (_STREAMS * _BLOCK_ROWS)
_LOG2E = 1.4426950408889634


def _block_stats(x):
    bm = jnp.max(x, axis=0, keepdims=True)
    bt = jnp.sum(x, axis=0, keepdims=True)
    e = jnp.exp2(x * _LOG2E - bm * _LOG2E)
    ones = jnp.ones((1, x.shape[0]), jnp.float32)
    bs = jnp.dot(ones, e, preferred_element_type=jnp.float32)
    return bm, bs, bt


def _merge(m1, s1, m2, s2):
    m = jnp.maximum(m1, m2)
    s = (s1 * jnp.exp2((m1 - m) * _LOG2E)
         + s2 * jnp.exp2((m2 - m) * _LOG2E))
    return m, s


def _listmle_body(*refs):
    x_refs = refs[:_STREAMS]
    out_ref, m_ref, s_ref, t_ref = refs[_STREAMS:]
    i = pl.program_id(0)

    stats = [_block_stats(x_ref[...]) for x_ref in x_refs]
    m_all, s_all, t_all = stats[0]
    for bm, bs, bt in stats[1:]:
        m_all, s_all = _merge(m_all, s_all, bm, bs)
        t_all = t_all + bt

    @pl.when(i == 0)
    def _init():
        m_ref[...] = m_all
        s_ref[...] = s_all
        t_ref[...] = t_all

    @pl.when(i > 0)
    def _update():
        m_new, s_new = _merge(m_ref[...], s_ref[...], m_all, s_all)
        m_ref[...] = m_new
        s_ref[...] = s_new
        t_ref[...] = t_ref[...] + t_all

    @pl.when(i == pl.num_programs(0) - 1)
    def _finish():
        lse = m_ref[...] + jnp.log(s_ref[...])
        out_ref[...] = (_ROWS * jnp.sum(lse, keepdims=True)
                        - jnp.sum(t_ref[...], keepdims=True))


@functools.partial(jax.jit, static_argnames=())
def _listmle_loss(scores):
    out = pl.pallas_call(
        _listmle_body,
        grid=(_GRID,),
        in_specs=[
            pl.BlockSpec((_BLOCK_ROWS, _COLS), lambda i, k=k: (i + k * _GRID, 0))
            for k in range(_STREAMS)
        ],
        out_specs=pl.BlockSpec((1, 1), lambda i: (0, 0)),
        out_shape=jax.ShapeDtypeStruct((1, 1), jnp.float32),
        scratch_shapes=[
            pltpu.VMEM((1, _COLS), jnp.float32),
            pltpu.VMEM((1, _COLS), jnp.float32),
            pltpu.VMEM((1, _COLS), jnp.float32),
        ],
    )(*([scores] * _STREAMS))
    return out[0, 0]


def kernel(scores, targets):
    del targets
    return _listmle_loss(scores)
